# baseline (speedup 1.0000x reference)
import jax
import jax.numpy as jnp
from jax.experimental import pallas as pl
from jax.experimental.pallas import tpu as pltpu

TOKENS = 8192
D_MODEL = 4096
UNITS = 64
TEMP = 0.5

BLOCK_M = 512
TILES = TOKENS // BLOCK_M
NBUF = 4


HALF = BLOCK_M // 2


def _router_kernel(x_hbm, w_ref, b_ref, o_ref, buf, sems):
    def copy(i, h):
        return pltpu.make_async_copy(
            x_hbm.at[pl.ds(i * BLOCK_M + h * HALF, HALF), :],
            buf.at[i % NBUF, pl.ds(h * HALF, HALF), :],
            sems.at[i % NBUF, h],
        )

    def start(i):
        copy(i, 0).start()
        copy(i, 1).start()

    for i in range(NBUF):
        start(i)
    for i in range(TILES):
        copy(i, 0).wait()
        copy(i, 1).wait()
        logits = jnp.dot(buf[i % NBUF], w_ref[...],
                         preferred_element_type=jnp.float32)
        logits = logits + b_ref[...]
        o_ref[pl.ds(i * BLOCK_M, BLOCK_M), :] = jax.nn.sigmoid(
            logits / (TEMP + 1e-08))
        if i + NBUF < TILES:
            start(i + NBUF)


def kernel(inputs, proj, logit_bias):
    bias2d = logit_bias.reshape(1, UNITS)
    return pl.pallas_call(
        _router_kernel,
        in_specs=[
            pl.BlockSpec(memory_space=pltpu.MemorySpace.HBM),
            pl.BlockSpec(memory_space=pltpu.MemorySpace.VMEM),
            pl.BlockSpec(memory_space=pltpu.MemorySpace.VMEM),
        ],
        out_specs=pl.BlockSpec(memory_space=pltpu.MemorySpace.VMEM),
        out_shape=jax.ShapeDtypeStruct((TOKENS, UNITS), jnp.float32),
        scratch_shapes=[
            pltpu.VMEM((NBUF, BLOCK_M, D_MODEL), jnp.float32),
            pltpu.SemaphoreType.DMA((NBUF, 2)),
        ],
        compiler_params=pltpu.CompilerParams(
            vmem_limit_bytes=100 * 1024 * 1024,
        ),
    )(inputs, proj, bias2d)


# speedup vs baseline: 1.0171x; 1.0171x over previous
import jax
import jax.numpy as jnp
from jax.experimental import pallas as pl
from jax.experimental.pallas import tpu as pltpu

TOKENS = 8192
D_MODEL = 4096
UNITS = 64
TEMP = 0.5

BLOCK_M = 512
TILES = TOKENS // BLOCK_M
NBUF = 4
NSUB = 8
SUB = BLOCK_M // NSUB


def _router_kernel(x_hbm, w_ref, b_ref, o_ref, buf, sems):
    def subcopy(i, c):
        return pltpu.make_async_copy(
            x_hbm.at[pl.ds(i * BLOCK_M + c * SUB, SUB), :],
            buf.at[i % NBUF, pl.ds(c * SUB, SUB), :],
            sems.at[i % NBUF],
        )

    def start(i):
        for c in range(NSUB):
            subcopy(i, c).start()

    def wait(i):
        for c in range(NSUB):
            subcopy(i, c).wait()

    for i in range(NBUF):
        start(i)
    for i in range(TILES):
        wait(i)
        logits = jnp.dot(buf[i % NBUF], w_ref[...],
                         preferred_element_type=jnp.float32)
        logits = logits + b_ref[...]
        o_ref[pl.ds(i * BLOCK_M, BLOCK_M), :] = jax.nn.sigmoid(
            logits / (TEMP + 1e-08))
        if i + NBUF < TILES:
            start(i + NBUF)


def kernel(inputs, proj, logit_bias):
    bias2d = logit_bias.reshape(1, UNITS)
    return pl.pallas_call(
        _router_kernel,
        in_specs=[
            pl.BlockSpec(memory_space=pltpu.MemorySpace.HBM),
            pl.BlockSpec(memory_space=pltpu.MemorySpace.VMEM),
            pl.BlockSpec(memory_space=pltpu.MemorySpace.VMEM),
        ],
        out_specs=pl.BlockSpec(memory_space=pltpu.MemorySpace.VMEM),
        out_shape=jax.ShapeDtypeStruct((TOKENS, UNITS), jnp.float32),
        scratch_shapes=[
            pltpu.VMEM((NBUF, BLOCK_M, D_MODEL), jnp.float32),
            pltpu.SemaphoreType.DMA((NBUF,)),
        ],
        compiler_params=pltpu.CompilerParams(
            vmem_limit_bytes=100 * 1024 * 1024,
        ),
    )(inputs, proj, bias2d)


# speedup vs baseline: 1.0901x; 1.0719x over previous
import jax
import jax.numpy as jnp
from jax.experimental import pallas as pl
from jax.experimental.pallas import tpu as pltpu

TOKENS = 8192
D_MODEL = 4096
UNITS = 64
TEMP = 0.5

BLOCK_M = 512


def _router_kernel(x_ref, w_ref, b_ref, o_ref):
    z = jnp.dot(x_ref[...], w_ref[...], preferred_element_type=jnp.float32)
    o_ref[...] = 0.5 * jnp.tanh(z + b_ref[...]) + 0.5


def kernel(inputs, proj, logit_bias):
    s = 0.5 / (TEMP + 1e-08)
    w2 = proj * s
    b2 = (logit_bias * s).reshape(1, UNITS)
    grid = (TOKENS // BLOCK_M,)
    return pl.pallas_call(
        _router_kernel,
        grid=grid,
        in_specs=[
            pl.BlockSpec((BLOCK_M, D_MODEL), lambda i: (i, 0)),
            pl.BlockSpec((D_MODEL, UNITS), lambda i: (0, 0)),
            pl.BlockSpec((1, UNITS), lambda i: (0, 0)),
        ],
        out_specs=pl.BlockSpec((BLOCK_M, UNITS), lambda i: (i, 0)),
        out_shape=jax.ShapeDtypeStruct((TOKENS, UNITS), jnp.float32),
        compiler_params=pltpu.CompilerParams(
            dimension_semantics=("arbitrary",),
        ),
    )(inputs, w2, b2)


# speedup vs baseline: 1.1079x; 1.0163x over previous
import jax
import jax.numpy as jnp
from jax.experimental import pallas as pl
from jax.experimental.pallas import tpu as pltpu

TOKENS = 8192
D_MODEL = 4096
UNITS = 64
TEMP = 0.5

BLOCK_M = 512


def _router_kernel(x_ref, w_ref, b_ref, o_ref):
    x = x_ref[...].astype(jnp.bfloat16)
    w = w_ref[...]
    z = jnp.dot(x, w, preferred_element_type=jnp.float32)
    o_ref[...] = 0.5 * jnp.tanh(z + b_ref[...]) + 0.5


def kernel(inputs, proj, logit_bias):
    s = 0.5 / (TEMP + 1e-08)
    w2 = (proj * s).astype(jnp.bfloat16)
    b2 = (logit_bias * s).reshape(1, UNITS)
    grid = (TOKENS // BLOCK_M,)
    return pl.pallas_call(
        _router_kernel,
        grid=grid,
        in_specs=[
            pl.BlockSpec((BLOCK_M, D_MODEL), lambda i: (i, 0)),
            pl.BlockSpec((D_MODEL, UNITS), lambda i: (0, 0)),
            pl.BlockSpec((1, UNITS), lambda i: (0, 0)),
        ],
        out_specs=pl.BlockSpec((BLOCK_M, UNITS), lambda i: (i, 0)),
        out_shape=jax.ShapeDtypeStruct((TOKENS, UNITS), jnp.float32),
        compiler_params=pltpu.CompilerParams(
            dimension_semantics=("arbitrary",),
        ),
    )(inputs, w2, b2)


# speedup vs baseline: 1.2083x; 1.0906x over previous
import jax
import jax.numpy as jnp
from jax.experimental import pallas as pl
from jax.experimental.pallas import tpu as pltpu

TOKENS = 8192
D_MODEL = 4096
UNITS = 64
TEMP = 0.5

BLOCK_M = 512


def _router_kernel(x_ref, w_ref, b_ref, o_ref):
    x = x_ref[...].astype(jnp.bfloat16)
    w = w_ref[...]
    z = jnp.dot(x, w, preferred_element_type=jnp.float32)
    o_ref[...] = 0.5 * jnp.tanh(z + b_ref[...]) + 0.5


def kernel(inputs, proj, logit_bias):
    s = 0.5 / (TEMP + 1e-08)
    w2 = (proj * s).astype(jnp.bfloat16)
    b2 = (logit_bias * s).reshape(1, UNITS)
    grid = (TOKENS // BLOCK_M,)
    return pl.pallas_call(
        _router_kernel,
        grid=grid,
        in_specs=[
            pl.BlockSpec((BLOCK_M, D_MODEL), lambda i: (i, 0)),
            pl.BlockSpec((D_MODEL, UNITS), lambda i: (0, 0)),
            pl.BlockSpec((1, UNITS), lambda i: (0, 0)),
        ],
        out_specs=pl.BlockSpec((BLOCK_M, UNITS), lambda i: (i, 0)),
        out_shape=jax.ShapeDtypeStruct((TOKENS, UNITS), jnp.float32),
        compiler_params=pltpu.CompilerParams(
            dimension_semantics=("arbitrary",),
            allow_input_fusion=[False, True, True],
        ),
    )(inputs, w2, b2)
